# Initial kernel scaffold; baseline (speedup 1.0000x reference)
import jax
import jax.numpy as jnp
from jax.experimental import pallas as pl


def kernel(positions, cos_cached, sin_cached):
    raise NotImplementedError("write your pallas kernel here")


# speedup vs baseline: 5.1581x; 5.1581x over previous
import functools

import jax
import jax.numpy as jnp
from jax import lax
from jax.experimental import pallas as pl
from jax.experimental.pallas import tpu as pltpu
from jax.experimental.pallas import tpu_sc as plsc

DIM = 128
NC = 2
NS = 16
NW = NC * NS
CHUNK = 128
NBUF = 2


def _sc_gather_body(pos_hbm, cos_hbm, sin_hbm, cos_out, sin_out,
                    idx_v, cbufs, sbufs, gsems, osems, n_chunks):
    wid = lax.axis_index("s") * NC + lax.axis_index("c")
    pltpu.sync_copy(pos_hbm.at[wid], idx_v)

    gathers = {}
    outs = {}
    for j in range(n_chunks + 1):
        if j < n_chunks:
            b = j % NBUF
            if j >= NBUF:
                outs[j - NBUF][0].wait()
                outs[j - NBUF][1].wait()
            gathers[j] = (
                pltpu.async_copy(cos_hbm.at[idx_v.at[j]], cbufs[b], gsems[2 * b]),
                pltpu.async_copy(sin_hbm.at[idx_v.at[j]], sbufs[b], gsems[2 * b + 1]),
            )
        if j >= 1:
            jj = j - 1
            b = jj % NBUF
            gathers[jj][0].wait()
            gathers[jj][1].wait()
            row0 = wid * (n_chunks * CHUNK) + jj * CHUNK
            outs[jj] = (
                pltpu.async_copy(cbufs[b], cos_out.at[pl.ds(row0, CHUNK)], osems[2 * b]),
                pltpu.async_copy(sbufs[b], sin_out.at[pl.ds(row0, CHUNK)], osems[2 * b + 1]),
            )
    for jj in range(max(n_chunks - NBUF, 0), n_chunks):
        outs[jj][0].wait()
        outs[jj][1].wait()


@functools.partial(jax.jit, static_argnames=())
def _rope_gather(pos3d, cos_cached, sin_cached):
    n_chunks = pos3d.shape[1]
    n_rows = NW * n_chunks * CHUNK
    mesh = plsc.VectorSubcoreMesh(core_axis_name="c", subcore_axis_name="s")
    scratch = (
        pltpu.VMEM((n_chunks, CHUNK), jnp.int32),
        [pltpu.VMEM((CHUNK, DIM), jnp.float32) for _ in range(NBUF)],
        [pltpu.VMEM((CHUNK, DIM), jnp.float32) for _ in range(NBUF)],
        [pltpu.SemaphoreType.DMA for _ in range(2 * NBUF)],
        [pltpu.SemaphoreType.DMA for _ in range(2 * NBUF)],
    )
    out_type = (
        jax.ShapeDtypeStruct((n_rows, DIM), jnp.float32),
        jax.ShapeDtypeStruct((n_rows, DIM), jnp.float32),
    )
    body = functools.partial(_sc_gather_body, n_chunks=n_chunks)
    return pl.kernel(
        body,
        out_type=out_type,
        mesh=mesh,
        scratch_types=scratch,
    )(pos3d, cos_cached, sin_cached)


def kernel(positions, cos_cached, sin_cached):
    batch, seq = positions.shape
    total = batch * seq
    n_chunks = total // (NW * CHUNK)
    pos3d = positions.reshape(NW, n_chunks, CHUNK)
    cos_flat, sin_flat = _rope_gather(pos3d, cos_cached, sin_cached)
    return (cos_flat.reshape(batch, seq, DIM), sin_flat.reshape(batch, seq, DIM))


# speedup vs baseline: 5.3011x; 1.0277x over previous
import functools

import jax
import jax.numpy as jnp
from jax import lax
from jax.experimental import pallas as pl
from jax.experimental.pallas import tpu as pltpu
from jax.experimental.pallas import tpu_sc as plsc

DIM = 128
NC = 2
NS = 16
NW = NC * NS
CHUNK = 128
NBUF = 3
DEPTH = 2


def _sc_gather_body(pos_hbm, cos_hbm, sin_hbm, cos_out, sin_out,
                    idx_v, cbufs, sbufs, gsems, osems, n_chunks):
    wid = lax.axis_index("s") * NC + lax.axis_index("c")
    pltpu.sync_copy(pos_hbm.at[wid], idx_v)

    gathers = {}
    outs = {}
    for j in range(n_chunks + DEPTH):
        if j < n_chunks:
            b = j % NBUF
            if j >= NBUF:
                outs[j - NBUF][0].wait()
                outs[j - NBUF][1].wait()
            gathers[j] = (
                pltpu.async_copy(cos_hbm.at[idx_v.at[j]], cbufs[b], gsems[2 * b]),
                pltpu.async_copy(sin_hbm.at[idx_v.at[j]], sbufs[b], gsems[2 * b + 1]),
            )
        if j >= DEPTH:
            jj = j - DEPTH
            b = jj % NBUF
            gathers[jj][0].wait()
            gathers[jj][1].wait()
            row0 = wid * (n_chunks * CHUNK) + jj * CHUNK
            outs[jj] = (
                pltpu.async_copy(cbufs[b], cos_out.at[pl.ds(row0, CHUNK)], osems[2 * b]),
                pltpu.async_copy(sbufs[b], sin_out.at[pl.ds(row0, CHUNK)], osems[2 * b + 1]),
            )
    for jj in range(max(n_chunks - NBUF, 0), n_chunks):
        outs[jj][0].wait()
        outs[jj][1].wait()


@functools.partial(jax.jit, static_argnames=())
def _rope_gather(pos3d, cos_cached, sin_cached):
    n_chunks = pos3d.shape[1]
    n_rows = NW * n_chunks * CHUNK
    mesh = plsc.VectorSubcoreMesh(core_axis_name="c", subcore_axis_name="s")
    scratch = (
        pltpu.VMEM((n_chunks, CHUNK), jnp.int32),
        [pltpu.VMEM((CHUNK, DIM), jnp.float32) for _ in range(NBUF)],
        [pltpu.VMEM((CHUNK, DIM), jnp.float32) for _ in range(NBUF)],
        [pltpu.SemaphoreType.DMA for _ in range(2 * NBUF)],
        [pltpu.SemaphoreType.DMA for _ in range(2 * NBUF)],
    )
    out_type = (
        jax.ShapeDtypeStruct((n_rows, DIM), jnp.float32),
        jax.ShapeDtypeStruct((n_rows, DIM), jnp.float32),
    )
    body = functools.partial(_sc_gather_body, n_chunks=n_chunks)
    return pl.kernel(
        body,
        out_type=out_type,
        mesh=mesh,
        scratch_types=scratch,
    )(pos3d, cos_cached, sin_cached)


def kernel(positions, cos_cached, sin_cached):
    batch, seq = positions.shape
    total = batch * seq
    n_chunks = total // (NW * CHUNK)
    pos3d = positions.reshape(NW, n_chunks, CHUNK)
    cos_flat, sin_flat = _rope_gather(pos3d, cos_cached, sin_cached)
    return (cos_flat.reshape(batch, seq, DIM), sin_flat.reshape(batch, seq, DIM))
